# Initial kernel scaffold; baseline (speedup 1.0000x reference)
import jax
import jax.numpy as jnp
from jax.experimental import pallas as pl


def kernel(emg_x, eeg_x, emg_edge_index, eeg_edge_index, emg_edge_attr, eeg_edge_attr, emg_w1, emg_b1, emg_w2, emg_b2, emg_w3, emg_b3, emg_w4, emg_b4, eeg_w1, eeg_b1, eeg_w2, eeg_b2, eeg_w3, eeg_b3, eeg_w4, eeg_b4, gat1_w, gat1_asrc, gat1_adst, gat1_b, gat2_w, gat2_asrc, gat2_adst, gat2_b):
    raise NotImplementedError("write your pallas kernel here")


# speedup vs baseline: 6.3067x; 6.3067x over previous
import functools

import jax
import jax.numpy as jnp
from jax import lax
from jax.experimental import pallas as pl
from jax.experimental.pallas import tpu as pltpu
from jax.experimental.pallas import tpu_sc as plsc

N = 10000
E = 160000
N2 = 2 * N

NC = 2
NS = 16
NW = NC * NS
L = 16

NP = 10240
N2P = 20480
EPT1 = E // NW
C1 = 1000
EPT2 = 10752
E2P = EPT2 * NW
C2 = 512
DC = 32

_SC_PARAMS = pltpu.CompilerParams(
    use_tc_tiling_on_sc=False, needs_layout_passes=False)


@functools.cache
def _mesh():
  return plsc.VectorSubcoreMesh(
      core_axis_name="c", subcore_axis_name="s", num_cores=NC,
      num_subcores=NS)


def _wid():
  return lax.axis_index("s") * NC + lax.axis_index("c")


def _fill2d(ref, nrows, ncols, val):
  v = jnp.full((L,), val, jnp.float32)

  def body(i, _):
    for k in range(ncols // L):
      ref[i, pl.ds(k * L, L)] = v
    return 0

  lax.fori_loop(0, nrows, body, 0)


def _fill1d(ref, n, val):
  v = jnp.full((L,), val, jnp.float32)

  def body(i, _):
    ref[pl.ds(i * L, L)] = v
    return 0

  lax.fori_loop(0, n // L, body, 0)


def _vshuffle(x, idx):
  return lax.gather(
      x, idx[:, None],
      dimension_numbers=lax.GatherDimensionNumbers(
          offset_dims=(), collapsed_slice_dims=(0,), start_index_map=(0,)),
      slice_sizes=(1,),
      mode=lax.GatherScatterMode.PROMISE_IN_BOUNDS)


def _seg_scan(keys, vals, op):
  iota = lax.iota(jnp.int32, L)
  v = vals
  for step in (1, 2, 4, 8):
    idx = jnp.maximum(iota - step, 0)
    kn = _vshuffle(keys, idx)
    vn = _vshuffle(v, idx)
    valid = (iota >= step) & (kn == keys)
    v = jnp.where(valid, op(v, vn), v)
  nxt = jnp.minimum(iota + 1, L - 1)
  kdn = _vshuffle(keys, nxt)
  is_last = (keys != kdn) | (iota == (L - 1))
  return v, is_last


@functools.partial(jax.jit, static_argnums=(3,))
def _gin_agg(x, src, dst, nch):
  nchunks = EPT1 // C1
  rows_per_tile = NP // NS

  def body(x_hbm, src_hbm, dst_hbm, out_hbm, sidx_v, didx_v, rows_v, buf_v,
           acc_sh, sem):
    cid = lax.axis_index("c")
    sid = lax.axis_index("s")
    wid = _wid()
    base = wid * EPT1
    for m in range(2):
      for cc in range(nch):
        _fill2d(buf_v, rows_per_tile, DC, 0.0)
        pltpu.sync_copy(
            buf_v, acc_sh.at[pl.ds(sid * rows_per_tile, rows_per_tile)])
        plsc.subcore_barrier()

        def chunk(ci, _):
          off = pl.multiple_of(base + ci * C1, 8)
          pltpu.sync_copy(src_hbm.at[m, pl.ds(off, C1)], sidx_v)
          pltpu.sync_copy(dst_hbm.at[m, pl.ds(off, C1)], didx_v)
          pltpu.async_copy(x_hbm.at[m, cc].at[sidx_v], rows_v, sem).wait()
          pltpu.sync_copy(rows_v, acc_sh.at[didx_v], add=True)
          return 0

        lax.fori_loop(0, nchunks, chunk, 0)
        plsc.subcore_barrier()
        pltpu.sync_copy(
            acc_sh.at[pl.ds(sid * rows_per_tile, rows_per_tile)], buf_v)
        pltpu.sync_copy(
            buf_v,
            out_hbm.at[m, cc, cid,
                       pl.ds(sid * rows_per_tile, rows_per_tile)])
        plsc.subcore_barrier()

  k = pl.kernel(
      body,
      out_type=jax.ShapeDtypeStruct((2, nch, NC, NP, DC), jnp.float32),
      mesh=_mesh(),
      compiler_params=_SC_PARAMS,
      scratch_types=[
          pltpu.VMEM((C1,), jnp.int32),
          pltpu.VMEM((C1,), jnp.int32),
          pltpu.VMEM((C1, DC), jnp.float32),
          pltpu.VMEM((rows_per_tile, DC), jnp.float32),
          pltpu.VMEM_SHARED((NP, DC), jnp.float32),
          pltpu.SemaphoreType.DMA,
      ])
  return k(x, src, dst)


@jax.jit
def _gat_alpha_amax(src, dst, a_src, a_dst):
  nchunks = EPT2 // C2
  NEG = -3.0e38

  def body(src_hbm, dst_hbm, asrc_hbm, adst_hbm, alpha_hbm, amax_hbm,
           asrc_v, adst_v, m_v, sidx_v, didx_v, alpha_v, sem):
    cid = lax.axis_index("c")
    sid = lax.axis_index("s")
    wid = _wid()
    pltpu.sync_copy(asrc_hbm, asrc_v)
    pltpu.sync_copy(adst_hbm, adst_v)
    _fill1d(m_v, N2P, NEG)
    base = wid * EPT2

    def chunk(ci, _):
      off = pl.multiple_of(base + ci * C2, 8)
      pltpu.sync_copy(src_hbm.at[pl.ds(off, C2)], sidx_v)
      pltpu.sync_copy(dst_hbm.at[pl.ds(off, C2)], didx_v)

      def vec(j, _):
        s16 = sidx_v[pl.ds(j * L, L)]
        d16 = didx_v[pl.ds(j * L, L)]
        a = (plsc.load_gather(asrc_v, [s16]) +
             plsc.load_gather(adst_v, [d16]))
        al = jnp.where(a >= 0, a, 0.2 * a)
        alpha_v[pl.ds(j * L, L)] = al
        dk, av = plsc.sort_key_val(d16, al)
        gm, is_last = _seg_scan(dk, av, jnp.maximum)
        cur = plsc.load_gather(m_v, [dk])
        plsc.store_scatter(m_v, [dk], jnp.maximum(cur, gm), mask=is_last)
        return 0

      lax.fori_loop(0, C2 // L, vec, 0)
      pltpu.sync_copy(alpha_v, alpha_hbm.at[pl.ds(off, C2)])
      return 0

    lax.fori_loop(0, nchunks, chunk, 0)
    pltpu.sync_copy(m_v, amax_hbm.at[cid, sid])

  k = pl.kernel(
      body,
      out_type=(jax.ShapeDtypeStruct((E2P,), jnp.float32),
                jax.ShapeDtypeStruct((NC, NS, N2P), jnp.float32)),
      mesh=_mesh(),
      compiler_params=_SC_PARAMS,
      scratch_types=[
          pltpu.VMEM((N2P,), jnp.float32),
          pltpu.VMEM((N2P,), jnp.float32),
          pltpu.VMEM((N2P,), jnp.float32),
          pltpu.VMEM((C2,), jnp.int32),
          pltpu.VMEM((C2,), jnp.int32),
          pltpu.VMEM((C2,), jnp.float32),
          pltpu.SemaphoreType.DMA,
      ])
  return k(src, dst, a_src, a_dst)


@jax.jit
def _gat_exp_denom(dst, alpha, amax):
  nchunks = EPT2 // C2

  def body(dst_hbm, alpha_hbm, amax_hbm, ex_hbm, den_hbm,
           am_v, d_v, didx_v, al_v, ex_v, sem):
    cid = lax.axis_index("c")
    sid = lax.axis_index("s")
    wid = _wid()
    pltpu.sync_copy(amax_hbm, am_v)
    _fill1d(d_v, N2P, 0.0)
    base = wid * EPT2

    def chunk(ci, _):
      off = pl.multiple_of(base + ci * C2, 8)
      pltpu.sync_copy(dst_hbm.at[pl.ds(off, C2)], didx_v)
      pltpu.sync_copy(alpha_hbm.at[pl.ds(off, C2)], al_v)

      def vec(j, _):
        d16 = didx_v[pl.ds(j * L, L)]
        al = al_v[pl.ds(j * L, L)]
        am = plsc.load_gather(am_v, [d16])
        e = jnp.exp(al - am)
        ex_v[pl.ds(j * L, L)] = e
        dk, ev = plsc.sort_key_val(d16, e)
        gs, is_last = _seg_scan(dk, ev, lambda a, b: a + b)
        cur = plsc.load_gather(d_v, [dk])
        plsc.store_scatter(d_v, [dk], cur + gs, mask=is_last)
        return 0

      lax.fori_loop(0, C2 // L, vec, 0)
      pltpu.sync_copy(ex_v, ex_hbm.at[pl.ds(off, C2)])
      return 0

    lax.fori_loop(0, nchunks, chunk, 0)
    pltpu.sync_copy(d_v, den_hbm.at[cid, sid])

  k = pl.kernel(
      body,
      out_type=(jax.ShapeDtypeStruct((E2P,), jnp.float32),
                jax.ShapeDtypeStruct((NC, NS, N2P), jnp.float32)),
      mesh=_mesh(),
      compiler_params=_SC_PARAMS,
      scratch_types=[
          pltpu.VMEM((N2P,), jnp.float32),
          pltpu.VMEM((N2P,), jnp.float32),
          pltpu.VMEM((C2,), jnp.int32),
          pltpu.VMEM((C2,), jnp.float32),
          pltpu.VMEM((C2,), jnp.float32),
          pltpu.SemaphoreType.DMA,
      ])
  return k(dst, alpha, amax)


@functools.partial(jax.jit, static_argnums=(4,))
def _gat_wscatter(h, src, dst, ex, nch):
  nchunks = EPT2 // C2
  seg = N2P // NS

  def body(h_hbm, src_hbm, dst_hbm, ex_hbm, out_hbm,
           sidx_v, didx_v, ex_v, rows_v, buf_v, acc_sh, sem):
    cid = lax.axis_index("c")
    sid = lax.axis_index("s")
    wid = _wid()
    base = wid * EPT2
    for cc in range(nch):
      _fill2d(buf_v, seg, DC, 0.0)
      pltpu.sync_copy(buf_v, acc_sh.at[pl.ds(sid * seg, seg)])
      plsc.subcore_barrier()

      def chunk(ci, _):
        off = pl.multiple_of(base + ci * C2, 8)
        pltpu.sync_copy(src_hbm.at[pl.ds(off, C2)], sidx_v)
        pltpu.sync_copy(dst_hbm.at[pl.ds(off, C2)], didx_v)
        pltpu.sync_copy(ex_hbm.at[pl.ds(off, C2)], ex_v)
        pltpu.async_copy(h_hbm.at[cc].at[sidx_v], rows_v, sem).wait()

        def scale(e, _):
          ev = plsc.load_gather(ex_v, [jnp.full((L,), e, jnp.int32)])
          for kk in range(DC // L):
            rows_v[e, pl.ds(kk * L, L)] = rows_v[e, pl.ds(kk * L, L)] * ev
          return 0

        lax.fori_loop(0, C2, scale, 0)
        pltpu.sync_copy(rows_v, acc_sh.at[didx_v], add=True)
        return 0

      lax.fori_loop(0, nchunks, chunk, 0)
      plsc.subcore_barrier()
      pltpu.sync_copy(acc_sh.at[pl.ds(sid * seg, seg)], buf_v)
      pltpu.sync_copy(buf_v, out_hbm.at[cc, cid, pl.ds(sid * seg, seg)])
      plsc.subcore_barrier()

  k = pl.kernel(
      body,
      out_type=jax.ShapeDtypeStruct((nch, NC, N2P, DC), jnp.float32),
      mesh=_mesh(),
      compiler_params=_SC_PARAMS,
      scratch_types=[
          pltpu.VMEM((C2,), jnp.int32),
          pltpu.VMEM((C2,), jnp.int32),
          pltpu.VMEM((C2,), jnp.float32),
          pltpu.VMEM((C2, DC), jnp.float32),
          pltpu.VMEM((seg, DC), jnp.float32),
          pltpu.VMEM_SHARED((N2P, DC), jnp.float32),
          pltpu.SemaphoreType.DMA,
      ])
  return k(h, src, dst, ex)


@functools.partial(jax.jit, static_argnums=(6,))
def _gin_mlp(x, p, w1, b1, w2, b2, relu_out):
  Dout = w2.shape[1]

  def body(x_ref, p_ref, w1_ref, b1_ref, w2_ref, b2_ref, o_ref):
    t = x_ref[...] + p_ref[0] + p_ref[1]
    h = jnp.maximum(
        jnp.dot(t, w1_ref[...], preferred_element_type=jnp.float32)
        + b1_ref[...], 0.0)
    o = jnp.dot(h, w2_ref[...], preferred_element_type=jnp.float32) \
        + b2_ref[...]
    if relu_out:
      o = jnp.maximum(o, 0.0)
    o_ref[...] = o

  return pl.pallas_call(
      body,
      out_shape=jax.ShapeDtypeStruct((NP, Dout), jnp.float32),
  )(x, p, w1, b1, w2, b2)


@jax.jit
def _gat_pre(x, w, asrc, adst):
  Dout = w.shape[1]
  nch = Dout // DC

  del nch

  def body(x_ref, w_ref, as_ref, ad_ref, h_ref, s_ref, d_ref):
    h = jnp.dot(x_ref[...], w_ref[...], preferred_element_type=jnp.float32)
    h_ref[...] = h
    s_ref[...] = jnp.dot(h, as_ref[...], preferred_element_type=jnp.float32)
    d_ref[...] = jnp.dot(h, ad_ref[...], preferred_element_type=jnp.float32)

  return pl.pallas_call(
      body,
      out_shape=(jax.ShapeDtypeStruct((N2P, Dout), jnp.float32),
                 jax.ShapeDtypeStruct((N2P, 1), jnp.float32),
                 jax.ShapeDtypeStruct((N2P, 1), jnp.float32)),
  )(x, w, asrc, adst)


@functools.partial(jax.jit, static_argnums=(1,))
def _combine(parts, is_max):
  def body(p_ref, o_ref):
    acc = p_ref[0, 0]
    for c in range(NC):
      for s in range(NS):
        if c == 0 and s == 0:
          continue
        if is_max:
          acc = jnp.maximum(acc, p_ref[c, s])
        else:
          acc = acc + p_ref[c, s]
    o_ref[...] = acc

  return pl.pallas_call(
      body,
      out_shape=jax.ShapeDtypeStruct((N2P,), jnp.float32),
  )(parts)


@functools.partial(jax.jit, static_argnums=(3,))
def _gat_post(acc, den, bias, relu_out):
  nch = acc.shape[0]
  Dtot = nch * DC
  R = 2048

  def body(a_ref, d_ref, b_ref, o_ref):
    d = d_ref[0] + 1e-16
    for c in range(nch):
      s = a_ref[c, 0] + a_ref[c, 1]
      y = s / d[:, None] + b_ref[:, c * DC:(c + 1) * DC]
      if relu_out:
        y = jnp.maximum(y, 0.0)
      o_ref[:, c * DC:(c + 1) * DC] = y

  return pl.pallas_call(
      body,
      grid=(N2P // R,),
      in_specs=[
          pl.BlockSpec((nch, NC, R, DC), lambda i: (0, 0, i, 0)),
          pl.BlockSpec((1, R), lambda i: (0, i)),
          pl.BlockSpec((1, Dtot), lambda i: (0, 0)),
      ],
      out_specs=pl.BlockSpec((R, Dtot), lambda i: (i, 0)),
      out_shape=jax.ShapeDtypeStruct((N2P, Dtot), jnp.float32),
  )(acc, den.reshape(1, N2P), bias)


def _split_cols(x, nch):
  r = x.shape[0]
  return x.reshape(r, nch, DC).transpose(1, 0, 2)


def _gat_block(x, src, dst, w, asrc, adst, bias, relu_out):
  h, a_s, a_d = _gat_pre(x, w, asrc, adst)
  nch = h.shape[1] // DC
  h_chunks = _split_cols(h, nch)
  alpha, amax_p = _gat_alpha_amax(src, dst, a_s[:, 0], a_d[:, 0])
  amax = _combine(amax_p, True)
  ex, den_p = _gat_exp_denom(dst, alpha, amax)
  den = _combine(den_p, False)
  acc = _gat_wscatter(h_chunks, src, dst, ex, nch)
  return _gat_post(acc, den, bias, relu_out)


def kernel(emg_x, eeg_x, emg_edge_index, eeg_edge_index, emg_edge_attr,
           eeg_edge_attr, emg_w1, emg_b1, emg_w2, emg_b2, emg_w3, emg_b3,
           emg_w4, emg_b4, eeg_w1, eeg_b1, eeg_w2, eeg_b2, eeg_w3, eeg_b3,
           eeg_w4, eeg_b4, gat1_w, gat1_asrc, gat1_adst, gat1_b, gat2_w,
           gat2_asrc, gat2_adst, gat2_b):
  f32 = jnp.float32
  ex_pad = jnp.pad(emg_x, ((0, NP - N), (0, DC - 10)))
  gx_pad = jnp.pad(eeg_x, ((0, NP - N), (0, DC - 10)))
  x0 = jnp.stack([ex_pad, gx_pad])[:, None]
  ew1 = jnp.pad(emg_w1, ((0, DC - 10), (0, 0)))
  gw1 = jnp.pad(eeg_w1, ((0, DC - 10), (0, 0)))
  src = jnp.stack([emg_edge_index[0], eeg_edge_index[0]])
  dst = jnp.stack([emg_edge_index[1], eeg_edge_index[1]])

  p0 = _gin_agg(x0, src, dst, 1)
  h_emg = _gin_mlp(ex_pad, p0[0, 0], ew1, emg_b1.reshape(1, -1), emg_w2,
                   emg_b2.reshape(1, -1), True)
  h_eeg = _gin_mlp(gx_pad, p0[1, 0], gw1, eeg_b1.reshape(1, -1), eeg_w2,
                   eeg_b2.reshape(1, -1), True)
  x1 = jnp.stack([_split_cols(h_emg, 2), _split_cols(h_eeg, 2)])
  p1 = _gin_agg(x1, src, dst, 2)
  p1 = jnp.concatenate([p1[:, 0], p1[:, 1]], axis=-1)
  emg_feat = _gin_mlp(h_emg, p1[0], emg_w3, emg_b3.reshape(1, -1), emg_w4,
                      emg_b4.reshape(1, -1), False)
  eeg_feat = _gin_mlp(h_eeg, p1[1], eeg_w3, eeg_b3.reshape(1, -1), eeg_w4,
                      eeg_b4.reshape(1, -1), False)

  x = jnp.concatenate(
      [emg_feat[:N], eeg_feat[:N],
       jnp.zeros((N2P - N2, 128), f32)])
  loop = jnp.arange(N2, dtype=jnp.int32)
  npad = E2P - (2 * E + N2)
  src2 = jnp.concatenate(
      [src[0], src[1] + N, loop, jnp.zeros((npad,), jnp.int32)])
  dst2 = jnp.concatenate(
      [dst[0], dst[1] + N, loop, jnp.full((npad,), N2, jnp.int32)])

  h1 = _gat_block(x, src2, dst2, gat1_w, gat1_asrc.reshape(-1, 1),
                  gat1_adst.reshape(-1, 1), gat1_b.reshape(1, -1), True)
  h2 = _gat_block(h1, src2, dst2, gat2_w, gat2_asrc.reshape(-1, 1),
                  gat2_adst.reshape(-1, 1), gat2_b.reshape(1, -1), False)
  return h2[:N2]
